# Initial kernel scaffold; baseline (speedup 1.0000x reference)
import jax
import jax.numpy as jnp
from jax.experimental import pallas as pl


def kernel(adj_rows, adj_cols, adj_vals, img_rows, img_cols, img_vals, txt_rows, txt_cols, txt_vals, uEmbeds, iEmbeds, image_trans, text_trans, image_embedding, text_embedding):
    raise NotImplementedError("write your pallas kernel here")


# speedup vs baseline: 3.2212x; 3.2212x over previous
import functools

import jax
import jax.numpy as jnp
from jax import lax
from jax.experimental import pallas as pl
from jax.experimental.pallas import tpu as pltpu
from jax.experimental.pallas import tpu_sc as plsc

USER = 5000
ITEM = 5000
N = USER + ITEM
E = 320000
D = 128
IMG_DIM = 4096
TXT_DIM = 768
LAM = 0.2

NCORE = 2
NSUB = 16
CHUNK = 128
CHUNKS_PER_TILE = -(-E // (NSUB * CHUNK))
EPAD = CHUNKS_PER_TILE * CHUNK * NSUB
NROW_TILE = 640
NPAD = NROW_TILE * NSUB


def _make_spmm(init_scale):
    has_init = init_scale is not None
    mesh = plsc.VectorSubcoreMesh(core_axis_name="c", subcore_axis_name="s")
    scratch = [
        pltpu.VMEM((CHUNK,), jnp.int32),
        pltpu.VMEM((CHUNK,), jnp.int32),
        pltpu.VMEM((CHUNK,), jnp.float32),
        pltpu.VMEM((CHUNK, D), jnp.float32),
    ]
    if has_init:
        scratch.append(pltpu.VMEM((CHUNK, D), jnp.float32))
    scratch.append(pltpu.VMEM_SHARED((NPAD, D), jnp.float32))

    def body(rows_hbm, cols_hbm, vals_hbm, x_hbm, *rest):
        if has_init:
            (ia_hbm, ib_hbm, out_hbm,
             cidx, ridx, vbuf, rowsbuf, binit, acc) = rest
        else:
            out_hbm, cidx, ridx, vbuf, rowsbuf, acc = rest
        c = lax.axis_index("c")
        s = lax.axis_index("s")
        row0 = s * NROW_TILE

        if has_init:
            off = 0
            while off < NROW_TILE:
                m = min(CHUNK, NROW_TILE - off)
                r0 = row0 + off
                pltpu.sync_copy(ia_hbm.at[c, pl.ds(r0, m)],
                                rowsbuf.at[pl.ds(0, m)])
                pltpu.sync_copy(ib_hbm.at[c, pl.ds(r0, m)],
                                binit.at[pl.ds(0, m)])

                def comb(r, carry):
                    for q in range(D // 16):
                        sl = pl.ds(q * 16, 16)
                        rowsbuf[r, sl] = rowsbuf[r, sl] + init_scale * binit[r, sl]
                    return carry

                lax.fori_loop(0, m, comb, 0)
                pltpu.sync_copy(rowsbuf.at[pl.ds(0, m)], acc.at[pl.ds(r0, m)])
                off += m
        else:
            def zero_row(r, carry):
                for q in range(D // 16):
                    rowsbuf[r, pl.ds(q * 16, 16)] = jnp.zeros((16,), jnp.float32)
                return carry

            lax.fori_loop(0, CHUNK, zero_row, 0)
            off = 0
            while off < NROW_TILE:
                m = min(CHUNK, NROW_TILE - off)
                pltpu.sync_copy(rowsbuf.at[pl.ds(0, m)],
                                acc.at[pl.ds(row0 + off, m)])
                off += m
        plsc.subcore_barrier()

        ebase = s * CHUNKS_PER_TILE * CHUNK

        def chunk_body(j, carry):
            eoff = ebase + j * CHUNK
            pltpu.sync_copy(cols_hbm.at[c, pl.ds(eoff, CHUNK)], cidx)
            pltpu.sync_copy(rows_hbm.at[c, pl.ds(eoff, CHUNK)], ridx)
            pltpu.sync_copy(vals_hbm.at[c, pl.ds(eoff, CHUNK)], vbuf)
            pltpu.sync_copy(x_hbm.at[c].at[cidx], rowsbuf)

            def scale(g, carry2):
                v16 = vbuf[pl.ds(g * 16, 16)]
                for e in range(16):
                    v = v16[e]
                    i = g * 16 + e
                    for q in range(D // 16):
                        sl = pl.ds(q * 16, 16)
                        rowsbuf[i, sl] = rowsbuf[i, sl] * v
                return carry2

            lax.fori_loop(0, CHUNK // 16, scale, 0)
            pltpu.sync_copy(rowsbuf, acc.at[ridx], add=True)
            return carry

        lax.fori_loop(0, CHUNKS_PER_TILE, chunk_body, 0)
        plsc.subcore_barrier()

        pltpu.sync_copy(acc.at[pl.ds(row0, NROW_TILE)],
                        out_hbm.at[c, pl.ds(row0, NROW_TILE)])

    return pl.kernel(
        body,
        out_type=jax.ShapeDtypeStruct((NCORE, NPAD, D), jnp.float32),
        mesh=mesh,
        scratch_types=scratch,
    )


def _modal_proj(K, bm):
    def body(a_ref, w_ref, o_ref):
        y = jnp.dot(a_ref[...], w_ref[...], preferred_element_type=jnp.float32)
        y = jnp.where(y >= 0, y, 0.2 * y)
        nrm = jnp.sqrt(jnp.sum(y * y, axis=1, keepdims=True))
        o_ref[...] = y / jnp.maximum(nrm, 1e-12)

    return pl.pallas_call(
        body,
        grid=(ITEM // bm,),
        in_specs=[pl.BlockSpec((bm, K), lambda i: (i, 0)),
                  pl.BlockSpec((K, D), lambda i: (0, 0))],
        out_specs=pl.BlockSpec((bm, D), lambda i: (i, 0)),
        out_shape=jax.ShapeDtypeStruct((ITEM, D), jnp.float32),
    )


def kernel(adj_rows, adj_cols, adj_vals, img_rows, img_cols, img_vals,
           txt_rows, txt_cols, txt_vals, uEmbeds, iEmbeds,
           image_trans, text_trans, image_embedding, text_embedding):
    padlen = EPAD - E

    def pad1(a):
        return jnp.concatenate([a, jnp.zeros((padlen,), a.dtype)])

    adj_r = pad1(adj_rows)
    adj_c = pad1(adj_cols)
    adj_v = pad1(adj_vals)
    adj_r2 = jnp.stack([adj_r, adj_r])
    adj_c2 = jnp.stack([adj_c, adj_c])
    adj_v2 = jnp.stack([adj_v, adj_v])
    mod_r2 = jnp.stack([pad1(img_rows), pad1(txt_rows)])
    mod_c2 = jnp.stack([pad1(img_cols), pad1(txt_cols)])
    mod_v2 = jnp.stack([pad1(img_vals), pad1(txt_vals)])

    n_img = _modal_proj(IMG_DIM, 200)(image_embedding, image_trans)
    n_txt = _modal_proj(TXT_DIM, 200)(text_embedding, text_trans)

    base = jnp.concatenate([uEmbeds, iEmbeds], axis=0)

    spmm = _make_spmm(None)
    spmm_lam = _make_spmm(LAM)
    spmm_one = _make_spmm(1.0)

    Z = spmm(mod_r2, mod_c2, mod_v2, jnp.stack([base, base]))
    xB = jnp.stack([jnp.concatenate([uEmbeds, n_img], axis=0),
                    jnp.concatenate([uEmbeds, n_txt], axis=0)])
    S1 = spmm(adj_r2, adj_c2, adj_v2, xB)
    xC = jnp.stack([jnp.concatenate([S1[0, :USER], iEmbeds], axis=0),
                    jnp.concatenate([S1[1, :USER], iEmbeds], axis=0)])
    E0 = spmm_lam(adj_r2, adj_c2, adj_v2, xC, S1, Z)
    E1 = spmm(adj_r2, adj_c2, adj_v2, E0)
    OUT = spmm_one(adj_r2, adj_c2, adj_v2, E1, E0, E1)

    embeds = jnp.concatenate([OUT[0, :N], OUT[1, :N]], axis=-1)
    return embeds[:USER], embeds[USER:]


# speedup vs baseline: 3.6061x; 1.1195x over previous
import jax
import jax.numpy as jnp
from jax import lax
from jax.experimental import pallas as pl
from jax.experimental.pallas import tpu as pltpu
from jax.experimental.pallas import tpu_sc as plsc

USER = 5000
ITEM = 5000
N = USER + ITEM
E = 320000
D = 128
IMG_DIM = 4096
TXT_DIM = 768
LAM = 0.2

NCORE = 2
NSUB = 16
CHUNK = 128
CHUNKS = 160
EPAD = CHUNKS * CHUNK * NSUB
IB = 8
NB = CHUNKS // IB
NROW_TILE = 640
NPAD = NROW_TILE * NSUB


def _make_spmm(init_scale):
    has_init = init_scale is not None
    mesh = plsc.VectorSubcoreMesh(core_axis_name="c", subcore_axis_name="s")
    scratch = [
        pltpu.VMEM((2, IB, CHUNK), jnp.int32),
        pltpu.VMEM((2, IB, CHUNK), jnp.int32),
        pltpu.VMEM((2, IB, CHUNK), jnp.float32),
        pltpu.VMEM((CHUNK, D), jnp.float32),
        pltpu.VMEM((CHUNK, D), jnp.float32),
        pltpu.SemaphoreType.DMA,
        pltpu.SemaphoreType.DMA,
        pltpu.SemaphoreType.DMA,
        pltpu.SemaphoreType.DMA,
        pltpu.VMEM_SHARED((NPAD, D), jnp.float32),
    ]

    def body(rows_hbm, cols_hbm, vals_hbm, x_hbm, *rest):
        if has_init:
            (ia_hbm, ib_hbm, out_hbm,
             cI, rI, vI, buf0, buf1, isem0, isem1, gsem0, gsem1, acc) = rest
        else:
            (out_hbm,
             cI, rI, vI, buf0, buf1, isem0, isem1, gsem0, gsem1, acc) = rest
        c = lax.axis_index("c")
        s = lax.axis_index("s")
        row0 = s * NROW_TILE
        bufs = (buf0, buf1)
        gsems = (gsem0, gsem1)
        isems = (isem0, isem1)
        cb = s * CHUNKS

        def load_idx(m, bank):
            pltpu.async_copy(cols_hbm.at[c, pl.ds(cb + IB * m, IB)],
                             cI.at[bank], isems[bank])
            pltpu.async_copy(rows_hbm.at[c, pl.ds(cb + IB * m, IB)],
                             rI.at[bank], isems[bank])
            pltpu.async_copy(vals_hbm.at[c, pl.ds(cb + IB * m, IB)],
                             vI.at[bank], isems[bank])

        def wait_idx(bank):
            pltpu.make_async_copy(cols_hbm.at[c, pl.ds(cb, IB)],
                                  cI.at[bank], isems[bank]).wait()
            pltpu.make_async_copy(rows_hbm.at[c, pl.ds(cb, IB)],
                                  rI.at[bank], isems[bank]).wait()
            pltpu.make_async_copy(vals_hbm.at[c, pl.ds(cb, IB)],
                                  vI.at[bank], isems[bank]).wait()

        load_idx(0, 0)

        if has_init:
            off = 0
            while off < NROW_TILE:
                m = min(CHUNK, NROW_TILE - off)
                r0 = row0 + off
                pltpu.sync_copy(ia_hbm.at[c, pl.ds(r0, m)],
                                buf0.at[pl.ds(0, m)])
                pltpu.sync_copy(ib_hbm.at[c, pl.ds(r0, m)],
                                buf1.at[pl.ds(0, m)])

                def comb(r, carry):
                    for q in range(D // 16):
                        sl = pl.ds(q * 16, 16)
                        buf0[r, sl] = buf0[r, sl] + init_scale * buf1[r, sl]
                    return carry

                lax.fori_loop(0, m, comb, 0)
                pltpu.sync_copy(buf0.at[pl.ds(0, m)], acc.at[pl.ds(r0, m)])
                off += m
        else:
            def zero_row(r, carry):
                for q in range(D // 16):
                    buf0[r, pl.ds(q * 16, 16)] = jnp.zeros((16,), jnp.float32)
                return carry

            lax.fori_loop(0, CHUNK, zero_row, 0)
            off = 0
            while off < NROW_TILE:
                m = min(CHUNK, NROW_TILE - off)
                pltpu.sync_copy(buf0.at[pl.ds(0, m)],
                                acc.at[pl.ds(row0 + off, m)])
                off += m
        plsc.subcore_barrier()

        def start_gather(bank, jj, gb):
            pltpu.async_copy(x_hbm.at[c].at[cI.at[bank, jj]], bufs[gb],
                             gsems[gb])

        def wait_gather(bank, jj, gb):
            pltpu.make_async_copy(x_hbm.at[c].at[cI.at[bank, jj]], bufs[gb],
                                  gsems[gb]).wait()

        def process(bank, jj, gb):
            buf = bufs[gb]

            def scale(g, carry):
                v16 = vI[bank, jj, pl.ds(g * 16, 16)]
                for e in range(16):
                    v = v16[e]
                    i = g * 16 + e
                    for q in range(D // 16):
                        sl = pl.ds(q * 16, 16)
                        buf[i, sl] = buf[i, sl] * v
                return carry

            lax.fori_loop(0, CHUNK // 16, scale, 0)
            pltpu.sync_copy(buf, acc.at[rI.at[bank, jj]], add=True)

        def batch(m, bank):
            wait_idx(bank)
            start_gather(bank, 0, 0)
            start_gather(bank, 1, 1)

            @pl.when(m < NB - 1)
            def _():
                load_idx(m + 1, 1 - bank)

            for jj in range(IB):
                gb = jj % 2
                wait_gather(bank, jj, gb)
                process(bank, jj, gb)
                if jj + 2 < IB:
                    start_gather(bank, jj + 2, gb)

        def pipe(m2, carry):
            batch(2 * m2, 0)
            batch(2 * m2 + 1, 1)
            return carry

        lax.fori_loop(0, NB // 2, pipe, 0)
        plsc.subcore_barrier()

        pltpu.sync_copy(acc.at[pl.ds(row0, NROW_TILE)],
                        out_hbm.at[c, pl.ds(row0, NROW_TILE)])

    return pl.kernel(
        body,
        out_type=jax.ShapeDtypeStruct((NCORE, NPAD, D), jnp.float32),
        mesh=mesh,
        scratch_types=scratch,
    )


def _modal_proj(K, bm):
    def body(a_ref, w_ref, o_ref):
        y = jnp.dot(a_ref[...], w_ref[...], preferred_element_type=jnp.float32)
        y = jnp.where(y >= 0, y, 0.2 * y)
        nrm = jnp.sqrt(jnp.sum(y * y, axis=1, keepdims=True))
        o_ref[...] = y / jnp.maximum(nrm, 1e-12)

    return pl.pallas_call(
        body,
        grid=(ITEM // bm,),
        in_specs=[pl.BlockSpec((bm, K), lambda i: (i, 0)),
                  pl.BlockSpec((K, D), lambda i: (0, 0))],
        out_specs=pl.BlockSpec((bm, D), lambda i: (i, 0)),
        out_shape=jax.ShapeDtypeStruct((ITEM, D), jnp.float32),
    )


def kernel(adj_rows, adj_cols, adj_vals, img_rows, img_cols, img_vals,
           txt_rows, txt_cols, txt_vals, uEmbeds, iEmbeds,
           image_trans, text_trans, image_embedding, text_embedding):
    padlen = EPAD - E

    def pad3(a):
        p = jnp.concatenate([a, jnp.zeros((padlen,), a.dtype)])
        return p.reshape(NSUB * CHUNKS, CHUNK)

    adj_r = pad3(adj_rows)
    adj_c = pad3(adj_cols)
    adj_v = pad3(adj_vals)
    adj_r2 = jnp.stack([adj_r, adj_r])
    adj_c2 = jnp.stack([adj_c, adj_c])
    adj_v2 = jnp.stack([adj_v, adj_v])
    mod_r2 = jnp.stack([pad3(img_rows), pad3(txt_rows)])
    mod_c2 = jnp.stack([pad3(img_cols), pad3(txt_cols)])
    mod_v2 = jnp.stack([pad3(img_vals), pad3(txt_vals)])

    n_img = _modal_proj(IMG_DIM, 200)(image_embedding, image_trans)
    n_txt = _modal_proj(TXT_DIM, 200)(text_embedding, text_trans)

    base = jnp.concatenate([uEmbeds, iEmbeds], axis=0)

    spmm = _make_spmm(None)
    spmm_lam = _make_spmm(LAM)
    spmm_one = _make_spmm(1.0)

    Z = spmm(mod_r2, mod_c2, mod_v2, jnp.stack([base, base]))
    xB = jnp.stack([jnp.concatenate([uEmbeds, n_img], axis=0),
                    jnp.concatenate([uEmbeds, n_txt], axis=0)])
    S1 = spmm(adj_r2, adj_c2, adj_v2, xB)
    xC = jnp.stack([jnp.concatenate([S1[0, :USER], iEmbeds], axis=0),
                    jnp.concatenate([S1[1, :USER], iEmbeds], axis=0)])
    E0 = spmm_lam(adj_r2, adj_c2, adj_v2, xC, S1, Z)
    E1 = spmm(adj_r2, adj_c2, adj_v2, E0)
    OUT = spmm_one(adj_r2, adj_c2, adj_v2, E1, E0, E1)

    embeds = jnp.concatenate([OUT[0, :N], OUT[1, :N]], axis=-1)
    return embeds[:USER], embeds[USER:]
